# baseline (speedup 1.0000x reference)
import functools

import jax
import jax.numpy as jnp
from jax.experimental import pallas as pl
from jax.experimental.pallas import tpu as pltpu

NUM_EXPERTS = 8
TOP_K = 2


def _router_kernel(x_ref, wr_ref, combine_ref):
    logits = jnp.dot(x_ref[...], wr_ref[...].T, preferred_element_type=jnp.float32)
    probs = jax.nn.sigmoid(logits)
    e_iota = jax.lax.broadcasted_iota(jnp.int32, probs.shape, 1)
    m1 = jnp.max(probs, axis=-1, keepdims=True)
    i1 = jnp.min(jnp.where(probs == m1, e_iota, NUM_EXPERTS), axis=-1, keepdims=True)
    masked = jnp.where(e_iota == i1, -jnp.inf, probs)
    m2 = jnp.max(masked, axis=-1, keepdims=True)
    i2 = jnp.min(jnp.where(masked == m2, e_iota, NUM_EXPERTS), axis=-1, keepdims=True)
    denom = m1 + m2
    combine = (jnp.where(e_iota == i1, m1, 0.0) + jnp.where(e_iota == i2, m2, 0.0)) / denom
    combine_ref[...] = combine


def _moe_kernel(x_ref, wg_ref, wu_ref, wd_ref, combine_ref, out_ref):
    e = pl.program_id(1)
    i = pl.program_id(2)
    x = x_ref[...]
    g = jnp.dot(x, wg_ref[0].T, preferred_element_type=jnp.float32)
    u = jnp.dot(x, wu_ref[0].T, preferred_element_type=jnp.float32)
    h = (g * jax.nn.sigmoid(g)) * u
    y = jnp.dot(h, wd_ref[0].T, preferred_element_type=jnp.float32)
    combine = combine_ref[...]
    lane = jax.lax.broadcasted_iota(jnp.int32, combine.shape, 1)
    scale = jnp.sum(jnp.where(lane == e, combine, 0.0), axis=1, keepdims=True)
    contrib = scale * y

    @pl.when(jnp.logical_and(e == 0, i == 0))
    def _():
        out_ref[...] = contrib

    @pl.when(jnp.logical_not(jnp.logical_and(e == 0, i == 0)))
    def _():
        out_ref[...] += contrib


@functools.partial(jax.jit, static_argnames=("interpret",))
def kernel(hidden_states, w_router, w_gate, w_up, w_down, interpret=False):
    M, H = hidden_states.shape
    E, I, _ = w_gate.shape

    combine = pl.pallas_call(
        _router_kernel,
        out_shape=jax.ShapeDtypeStruct((M, E), jnp.float32),
        interpret=interpret,
    )(hidden_states, w_router)

    BM = 512
    BI = 128
    grid = (M // BM, E, I // BI)
    out = pl.pallas_call(
        _moe_kernel,
        grid=grid,
        in_specs=[
            pl.BlockSpec((BM, H), lambda m, e, i: (m, 0)),
            pl.BlockSpec((1, BI, H), lambda m, e, i: (e, i, 0)),
            pl.BlockSpec((1, BI, H), lambda m, e, i: (e, i, 0)),
            pl.BlockSpec((1, H, BI), lambda m, e, i: (e, 0, i)),
            pl.BlockSpec((BM, E), lambda m, e, i: (m, 0)),
        ],
        out_specs=pl.BlockSpec((BM, H), lambda m, e, i: (m, 0)),
        out_shape=jax.ShapeDtypeStruct((M, H), jnp.float32),
        compiler_params=pltpu.CompilerParams(
            dimension_semantics=("parallel", "arbitrary", "arbitrary"),
        ),
        interpret=interpret,
    )(hidden_states, w_gate, w_up, w_down, combine)
    return out


# speedup vs baseline: 1.6412x; 1.6412x over previous
import functools

import jax
import jax.numpy as jnp
from jax.experimental import pallas as pl
from jax.experimental.pallas import tpu as pltpu

NUM_EXPERTS = 8
TOP_K = 2


def _router_kernel(x_ref, wr_ref, combine_ref):
    logits = jnp.dot(x_ref[...], wr_ref[...].T, preferred_element_type=jnp.float32)
    probs = jax.nn.sigmoid(logits)
    e_iota = jax.lax.broadcasted_iota(jnp.int32, probs.shape, 1)
    m1 = jnp.max(probs, axis=-1, keepdims=True)
    i1 = jnp.min(jnp.where(probs == m1, e_iota, NUM_EXPERTS), axis=-1, keepdims=True)
    masked = jnp.where(e_iota == i1, -jnp.inf, probs)
    m2 = jnp.max(masked, axis=-1, keepdims=True)
    i2 = jnp.min(jnp.where(masked == m2, e_iota, NUM_EXPERTS), axis=-1, keepdims=True)
    denom = m1 + m2
    combine = (jnp.where(e_iota == i1, m1, 0.0) + jnp.where(e_iota == i2, m2, 0.0)) / denom
    combine_ref[...] = combine


def _moe_kernel(x_ref, wg_ref, wu_ref, wd_ref, combine_ref, out_ref):
    e = pl.program_id(1)
    x = x_ref[...]
    g = jnp.dot(x, wg_ref[0].T, preferred_element_type=jnp.float32)
    u = jnp.dot(x, wu_ref[0].T, preferred_element_type=jnp.float32)
    h = ((g * jax.nn.sigmoid(g)) * u).astype(jnp.bfloat16)
    y = jnp.dot(h, wd_ref[0].T, preferred_element_type=jnp.float32)
    combine = combine_ref[...]
    lane = jax.lax.broadcasted_iota(jnp.int32, combine.shape, 1)
    scale = jnp.sum(jnp.where(lane == e, combine, 0.0), axis=1, keepdims=True)
    contrib = scale * y

    @pl.when(e == 0)
    def _():
        out_ref[...] = contrib

    @pl.when(e != 0)
    def _():
        out_ref[...] += contrib


@functools.partial(jax.jit, static_argnames=("interpret",))
def kernel(hidden_states, w_router, w_gate, w_up, w_down, interpret=False):
    M, H = hidden_states.shape
    E, I, _ = w_gate.shape

    combine = pl.pallas_call(
        _router_kernel,
        out_shape=jax.ShapeDtypeStruct((M, E), jnp.float32),
        interpret=interpret,
    )(hidden_states, w_router)

    x_bf = hidden_states.astype(jnp.bfloat16)
    wg_bf = w_gate.astype(jnp.bfloat16)
    wu_bf = w_up.astype(jnp.bfloat16)
    wd_bf = w_down.astype(jnp.bfloat16)

    BM = 512
    grid = (M // BM, E)
    out = pl.pallas_call(
        _moe_kernel,
        grid=grid,
        in_specs=[
            pl.BlockSpec((BM, H), lambda m, e: (m, 0)),
            pl.BlockSpec((1, I, H), lambda m, e: (e, 0, 0)),
            pl.BlockSpec((1, I, H), lambda m, e: (e, 0, 0)),
            pl.BlockSpec((1, H, I), lambda m, e: (e, 0, 0)),
            pl.BlockSpec((BM, E), lambda m, e: (m, 0)),
        ],
        out_specs=pl.BlockSpec((BM, H), lambda m, e: (m, 0)),
        out_shape=jax.ShapeDtypeStruct((M, H), jnp.float32),
        compiler_params=pltpu.CompilerParams(
            dimension_semantics=("parallel", "arbitrary"),
        ),
        interpret=interpret,
    )(x_bf, wg_bf, wu_bf, wd_bf, combine)
    return out


# speedup vs baseline: 2.1821x; 1.3296x over previous
import functools

import jax
import jax.numpy as jnp
from jax import lax
from jax.experimental import pallas as pl
from jax.experimental.pallas import tpu as pltpu
from jax.experimental.pallas import tpu_sc as plsc

NUM_EXPERTS = 8
TOP_K = 2
BM = 256
_M = 2048
NBLK = (_M * TOP_K) // BM + NUM_EXPERTS - 1
NR = NBLK * BM
BE_PAD = 128
NW = 32
KC = 4
CH = 32
_CS = 512


def _router_kernel(x_ref, wr_ref, dest_ref, pw_ref, be_ref):
    E = NUM_EXPERTS
    M = x_ref.shape[0]
    logits = lax.dot_general(
        wr_ref[...], x_ref[...], (((1,), (1,)), ((), ())),
        preferred_element_type=jnp.float32)
    probs = jax.nn.sigmoid(logits)
    sub = lax.broadcasted_iota(jnp.int32, (E, M), 0)
    m1 = jnp.max(probs, axis=0, keepdims=True)
    i1 = jnp.min(jnp.where(probs == m1, sub, E), axis=0, keepdims=True)
    masked = jnp.where(sub == i1, -1.0, probs)
    m2 = jnp.max(masked, axis=0, keepdims=True)
    i2 = jnp.min(jnp.where(masked == m2, sub, E), axis=0, keepdims=True)
    denom = m1 + m2
    pw_ref[0:1, :] = m1 / denom
    pw_ref[1:2, :] = m2 / denom

    oh1 = (sub == i1).astype(jnp.float32)
    oh2 = (sub == i2).astype(jnp.float32)
    r_io = lax.broadcasted_iota(jnp.int32, (_CS, _CS), 0)
    c_io = lax.broadcasted_iota(jnp.int32, (_CS, _CS), 1)
    upper = (r_io < c_io).astype(jnp.float32)

    def excl_cumsum(oh, tot):
        parts = []
        for c in range(M // _CS):
            blk = oh[:, c * _CS:(c + 1) * _CS]
            rc = lax.dot_general(blk, upper, (((1,), (0,)), ((), ())),
                                 preferred_element_type=jnp.float32,
                                 precision=lax.Precision.HIGHEST) + tot
            parts.append(rc)
            tot = tot + jnp.sum(blk, axis=1, keepdims=True)
        return jnp.concatenate(parts, axis=1), tot

    zero = jnp.zeros((E, 1), jnp.float32)
    r1, tot1 = excl_cumsum(oh1, zero)
    r2, counts = excl_cumsum(oh2, tot1)
    padded = jnp.floor((counts + (BM - 1)) / BM) * BM
    er_io = lax.broadcasted_iota(jnp.int32, (E, E), 0)
    ec_io = lax.broadcasted_iota(jnp.int32, (E, E), 1)
    lower = (ec_io < er_io).astype(jnp.float32)
    offs = lax.dot_general(lower, padded, (((1,), (0,)), ((), ())),
                           preferred_element_type=jnp.float32,
                           precision=lax.Precision.HIGHEST)
    dest1 = jnp.sum(oh1 * (offs + r1), axis=0, keepdims=True)
    dest2 = jnp.sum(oh2 * (offs + r2), axis=0, keepdims=True)
    dest_ref[0:1, :] = dest1.astype(jnp.int32)
    dest_ref[1:2, :] = dest2.astype(jnp.int32)

    b_io = lax.broadcasted_iota(jnp.int32, (1, BE_PAD), 1).astype(jnp.float32)
    end_blk = (offs + padded) / BM
    esel = lax.broadcasted_iota(jnp.int32, (E, 1), 0)
    be = jnp.zeros((1, BE_PAD), jnp.float32)
    for e in range(E):
        eb_e = jnp.sum(jnp.where(esel == e, end_blk, 0.0), axis=0, keepdims=True)
        be = be + (b_io >= eb_e).astype(jnp.float32)
    be_ref[...] = jnp.minimum(be, E - 1).astype(jnp.int32)


def _gmm_kernel(be_sref, xs_ref, wg_ref, wu_ref, wd_ref, ys_ref):
    x = xs_ref[...].astype(jnp.bfloat16)
    g = jnp.dot(x, wg_ref[0].T, preferred_element_type=jnp.float32)
    u = jnp.dot(x, wu_ref[0].T, preferred_element_type=jnp.float32)
    h = ((g * jax.nn.sigmoid(g)) * u).astype(jnp.bfloat16)
    ys_ref[...] = jnp.dot(h, wd_ref[0].T, preferred_element_type=jnp.float32)


def _combine_kernel(g1_ref, g2_ref, pwt_ref, out_ref):
    pwt = pwt_ref[...]
    out_ref[...] = pwt[:, 0:1] * g1_ref[...] + pwt[:, 1:2] * g2_ref[...]


def _make_scatter(H, dtype):
    @functools.partial(
        pl.kernel,
        mesh=plsc.VectorSubcoreMesh(core_axis_name="c", subcore_axis_name="s"),
        out_type=jax.ShapeDtypeStruct((NR, H), dtype),
        scratch_types=[
            pltpu.VMEM((KC, CH), jnp.int32),
            pltpu.VMEM((CH, H), dtype),
            pltpu.SemaphoreType.DMA,
        ],
    )
    def scatter_k(x_hbm, idx_hbm, xs_hbm, idx_v, buf, sem):
        wid = lax.axis_index("s") * 2 + lax.axis_index("c")
        t0 = (wid % 16) * (KC * CH)
        pltpu.sync_copy(idx_hbm.at[wid], idx_v)
        for j in range(KC):
            pltpu.sync_copy(x_hbm.at[pl.ds(t0 + j * CH, CH)], buf)
            pltpu.async_copy(buf, xs_hbm.at[idx_v.at[j]], sem).wait()

    return scatter_k


def _make_gather(H, dtype):
    @functools.partial(
        pl.kernel,
        mesh=plsc.VectorSubcoreMesh(core_axis_name="c", subcore_axis_name="s"),
        out_type=jax.ShapeDtypeStruct((NW * KC * CH, H), dtype),
        scratch_types=[
            pltpu.VMEM((KC, CH), jnp.int32),
            pltpu.VMEM((CH, H), dtype),
            pltpu.SemaphoreType.DMA,
        ],
    )
    def gather_k(ys_hbm, idx_hbm, g_hbm, idx_v, buf, sem):
        wid = lax.axis_index("s") * 2 + lax.axis_index("c")
        p0 = wid * (KC * CH)
        pltpu.sync_copy(idx_hbm.at[wid], idx_v)
        for j in range(KC):
            pltpu.async_copy(ys_hbm.at[idx_v.at[j]], buf, sem).wait()
            pltpu.sync_copy(buf, g_hbm.at[pl.ds(p0 + j * CH, CH)])

    return gather_k


def kernel(hidden_states, w_router, w_gate, w_up, w_down):
    M, H = hidden_states.shape
    E, I, _ = w_gate.shape

    dest, pw, be = pl.pallas_call(
        _router_kernel,
        out_shape=(
            jax.ShapeDtypeStruct((TOP_K, M), jnp.int32),
            jax.ShapeDtypeStruct((TOP_K, M), jnp.float32),
            jax.ShapeDtypeStruct((1, BE_PAD), jnp.int32),
        ),
    )(hidden_states, w_router)

    idx3 = dest.reshape(NW, KC, CH)
    xs = _make_scatter(H, jnp.float32)(hidden_states, idx3)

    wg_bf = w_gate.astype(jnp.bfloat16)
    wu_bf = w_up.astype(jnp.bfloat16)
    wd_bf = w_down.astype(jnp.bfloat16)
    ys = pl.pallas_call(
        _gmm_kernel,
        grid_spec=pltpu.PrefetchScalarGridSpec(
            num_scalar_prefetch=1,
            grid=(NBLK,),
            in_specs=[
                pl.BlockSpec((BM, H), lambda b, be_ref: (b, 0)),
                pl.BlockSpec((1, I, H), lambda b, be_ref: (be_ref[b], 0, 0)),
                pl.BlockSpec((1, I, H), lambda b, be_ref: (be_ref[b], 0, 0)),
                pl.BlockSpec((1, H, I), lambda b, be_ref: (be_ref[b], 0, 0)),
            ],
            out_specs=pl.BlockSpec((BM, H), lambda b, be_ref: (b, 0)),
        ),
        out_shape=jax.ShapeDtypeStruct((NR, H), jnp.float32),
    )(be.reshape(BE_PAD), xs, wg_bf, wu_bf, wd_bf)

    g = _make_gather(H, jnp.float32)(ys, idx3)

    BT = 512
    out = pl.pallas_call(
        _combine_kernel,
        grid=(M // BT,),
        in_specs=[
            pl.BlockSpec((BT, H), lambda t: (t, 0)),
            pl.BlockSpec((BT, H), lambda t: (t + M // BT, 0)),
            pl.BlockSpec((BT, TOP_K), lambda t: (t, 0)),
        ],
        out_specs=pl.BlockSpec((BT, H), lambda t: (t, 0)),
        out_shape=jax.ShapeDtypeStruct((M, H), jnp.float32),
    )(g, g, pw.T)
    return out


# speedup vs baseline: 2.6191x; 1.2002x over previous
import functools

import jax
import jax.numpy as jnp
from jax import lax
from jax.experimental import pallas as pl
from jax.experimental.pallas import tpu as pltpu
from jax.experimental.pallas import tpu_sc as plsc

NUM_EXPERTS = 8
TOP_K = 2
BM = 256
_M = 2048
NBLK = (_M * TOP_K) // BM + NUM_EXPERTS - 1
NR = NBLK * BM
BE_PAD = 128
NW = 32
KC = 4
CH = 32
_CS = 512


def _router_kernel(x_ref, wr_ref, dest_ref, pw_ref, be_ref):
    E = NUM_EXPERTS
    M = x_ref.shape[0]
    logits = lax.dot_general(
        wr_ref[...], x_ref[...], (((1,), (1,)), ((), ())),
        preferred_element_type=jnp.float32)
    probs = jax.nn.sigmoid(logits)
    sub = lax.broadcasted_iota(jnp.int32, (E, M), 0)
    m1 = jnp.max(probs, axis=0, keepdims=True)
    i1 = jnp.min(jnp.where(probs == m1, sub, E), axis=0, keepdims=True)
    masked = jnp.where(sub == i1, -1.0, probs)
    m2 = jnp.max(masked, axis=0, keepdims=True)
    i2 = jnp.min(jnp.where(masked == m2, sub, E), axis=0, keepdims=True)
    denom = m1 + m2
    pw_ref[0:1, :] = m1 / denom
    pw_ref[1:2, :] = m2 / denom

    oh1 = (sub == i1).astype(jnp.float32)
    oh2 = (sub == i2).astype(jnp.float32)
    r_io = lax.broadcasted_iota(jnp.int32, (_CS, _CS), 0)
    c_io = lax.broadcasted_iota(jnp.int32, (_CS, _CS), 1)
    upper = (r_io < c_io).astype(jnp.float32)

    def excl_cumsum(oh, tot):
        parts = []
        for c in range(M // _CS):
            blk = oh[:, c * _CS:(c + 1) * _CS]
            rc = lax.dot_general(blk, upper, (((1,), (0,)), ((), ())),
                                 preferred_element_type=jnp.float32,
                                 precision=lax.Precision.HIGHEST) + tot
            parts.append(rc)
            tot = tot + jnp.sum(blk, axis=1, keepdims=True)
        return jnp.concatenate(parts, axis=1), tot

    zero = jnp.zeros((E, 1), jnp.float32)
    r1, tot1 = excl_cumsum(oh1, zero)
    r2, counts = excl_cumsum(oh2, tot1)
    padded = jnp.floor((counts + (BM - 1)) / BM) * BM
    er_io = lax.broadcasted_iota(jnp.int32, (E, E), 0)
    ec_io = lax.broadcasted_iota(jnp.int32, (E, E), 1)
    lower = (ec_io < er_io).astype(jnp.float32)
    offs = lax.dot_general(lower, padded, (((1,), (0,)), ((), ())),
                           preferred_element_type=jnp.float32,
                           precision=lax.Precision.HIGHEST)
    dest1 = jnp.sum(oh1 * (offs + r1), axis=0, keepdims=True)
    dest2 = jnp.sum(oh2 * (offs + r2), axis=0, keepdims=True)
    dest_ref[0:1, :] = dest1.astype(jnp.int32)
    dest_ref[1:2, :] = dest2.astype(jnp.int32)

    b_io = lax.broadcasted_iota(jnp.int32, (1, BE_PAD), 1).astype(jnp.float32)
    end_blk = (offs + padded) / BM
    esel = lax.broadcasted_iota(jnp.int32, (E, 1), 0)
    be = jnp.zeros((1, BE_PAD), jnp.float32)
    for e in range(E):
        eb_e = jnp.sum(jnp.where(esel == e, end_blk, 0.0), axis=0, keepdims=True)
        be = be + (b_io >= eb_e).astype(jnp.float32)
    be_ref[...] = jnp.minimum(be, E - 1).astype(jnp.int32)


def _gateup_kernel(be_sref, xs_ref, wg_ref, wu_ref, h_ref):
    x = xs_ref[...].astype(jnp.bfloat16)
    wg = wg_ref[0].astype(jnp.bfloat16)
    wu = wu_ref[0].astype(jnp.bfloat16)
    g = jnp.dot(x, wg.T, preferred_element_type=jnp.float32)
    u = jnp.dot(x, wu.T, preferred_element_type=jnp.float32)
    h_ref[...] = ((g * jax.nn.sigmoid(g)) * u).astype(jnp.bfloat16)


def _down_kernel(be_sref, h_ref, wd_ref, ys_ref):
    wd = wd_ref[0].astype(jnp.bfloat16)
    ys_ref[...] = jnp.dot(h_ref[...], wd.T, preferred_element_type=jnp.float32)


def _combine_kernel(g1_ref, g2_ref, pwt_ref, out_ref):
    pwt = pwt_ref[...]
    out_ref[...] = pwt[:, 0:1] * g1_ref[...] + pwt[:, 1:2] * g2_ref[...]


def _make_scatter(H, dtype):
    @functools.partial(
        pl.kernel,
        mesh=plsc.VectorSubcoreMesh(core_axis_name="c", subcore_axis_name="s"),
        out_type=jax.ShapeDtypeStruct((NR, H), dtype),
        scratch_types=[
            pltpu.VMEM((KC, CH), jnp.int32),
            pltpu.VMEM((CH, H), dtype),
            pltpu.SemaphoreType.DMA,
        ],
    )
    def scatter_k(x_hbm, idx_hbm, xs_hbm, idx_v, buf, sem):
        wid = lax.axis_index("s") * 2 + lax.axis_index("c")
        t0 = (wid % 16) * (KC * CH)
        pltpu.sync_copy(idx_hbm.at[wid], idx_v)
        for j in range(KC):
            pltpu.sync_copy(x_hbm.at[pl.ds(t0 + j * CH, CH)], buf)
            pltpu.async_copy(buf, xs_hbm.at[idx_v.at[j]], sem).wait()

    return scatter_k


def _make_gather(H, dtype):
    @functools.partial(
        pl.kernel,
        mesh=plsc.VectorSubcoreMesh(core_axis_name="c", subcore_axis_name="s"),
        out_type=jax.ShapeDtypeStruct((NW * KC * CH, H), dtype),
        scratch_types=[
            pltpu.VMEM((KC, CH), jnp.int32),
            pltpu.VMEM((CH, H), dtype),
            pltpu.SemaphoreType.DMA,
        ],
    )
    def gather_k(ys_hbm, idx_hbm, g_hbm, idx_v, buf, sem):
        wid = lax.axis_index("s") * 2 + lax.axis_index("c")
        p0 = wid * (KC * CH)
        pltpu.sync_copy(idx_hbm.at[wid], idx_v)
        for j in range(KC):
            pltpu.async_copy(ys_hbm.at[idx_v.at[j]], buf, sem).wait()
            pltpu.sync_copy(buf, g_hbm.at[pl.ds(p0 + j * CH, CH)])

    return gather_k


def kernel(hidden_states, w_router, w_gate, w_up, w_down):
    M, H = hidden_states.shape
    E, I, _ = w_gate.shape

    dest, pw, be = pl.pallas_call(
        _router_kernel,
        out_shape=(
            jax.ShapeDtypeStruct((TOP_K, M), jnp.int32),
            jax.ShapeDtypeStruct((TOP_K, M), jnp.float32),
            jax.ShapeDtypeStruct((1, BE_PAD), jnp.int32),
        ),
    )(hidden_states, w_router)

    idx3 = dest.reshape(NW, KC, CH)
    xs = _make_scatter(H, jnp.float32)(hidden_states, idx3)

    be_flat = be.reshape(BE_PAD)
    h = pl.pallas_call(
        _gateup_kernel,
        grid_spec=pltpu.PrefetchScalarGridSpec(
            num_scalar_prefetch=1,
            grid=(NBLK,),
            in_specs=[
                pl.BlockSpec((BM, H), lambda b, be_ref: (b, 0)),
                pl.BlockSpec((1, I, H), lambda b, be_ref: (be_ref[b], 0, 0)),
                pl.BlockSpec((1, I, H), lambda b, be_ref: (be_ref[b], 0, 0)),
            ],
            out_specs=pl.BlockSpec((BM, I), lambda b, be_ref: (b, 0)),
        ),
        out_shape=jax.ShapeDtypeStruct((NR, I), jnp.bfloat16),
    )(be_flat, xs, w_gate, w_up)
    ys = pl.pallas_call(
        _down_kernel,
        grid_spec=pltpu.PrefetchScalarGridSpec(
            num_scalar_prefetch=1,
            grid=(NBLK,),
            in_specs=[
                pl.BlockSpec((BM, I), lambda b, be_ref: (b, 0)),
                pl.BlockSpec((1, H, I), lambda b, be_ref: (be_ref[b], 0, 0)),
            ],
            out_specs=pl.BlockSpec((BM, H), lambda b, be_ref: (b, 0)),
        ),
        out_shape=jax.ShapeDtypeStruct((NR, H), jnp.float32),
    )(be_flat, h, w_down)

    g = _make_gather(H, jnp.float32)(ys, idx3)

    BT = 512
    out = pl.pallas_call(
        _combine_kernel,
        grid=(M // BT,),
        in_specs=[
            pl.BlockSpec((BT, H), lambda t: (t, 0)),
            pl.BlockSpec((BT, H), lambda t: (t + M // BT, 0)),
            pl.BlockSpec((BT, TOP_K), lambda t: (t, 0)),
        ],
        out_specs=pl.BlockSpec((BT, H), lambda t: (t, 0)),
        out_shape=jax.ShapeDtypeStruct((M, H), jnp.float32),
    )(g, g, pw.T)
    return out
